# baseline (speedup 1.0000x reference)
import jax
import jax.numpy as jnp
from jax import lax
from jax.experimental import pallas as pl
from jax.experimental.pallas import tpu as pltpu
from jax.experimental.pallas import tpu_sc as plsc

B = 16384
K = 64
N = 1000000
BN = 16384
NW = 32
BPW = B // NW
NCH = 4
CH = BPW // NCH
L = 16

CB = 128
SC_BLKS = 80
NS = NW * SC_BLKS * CB
NT = N - NS
TC_OFF = NS // BN
assert NS % BN == 0


def _sc_scores_body(ut_hbm, it_hbm, wbt_hbm, su_hbm, si_hbm,
                    buf0, buf1, wbt_v, out_v, sem0, sem1):
    c = lax.axis_index("c")
    s = lax.axis_index("s")
    wid = s * 2 + c
    base = wid * SC_BLKS * CB

    pltpu.sync_copy(wbt_hbm, wbt_v)

    def table_loop(src_hbm, out_hbm, koff):
        pltpu.async_copy(src_hbm.at[:, pl.ds(base, CB)], buf0, sem0)
        pltpu.async_copy(src_hbm.at[:, pl.ds(base + CB, CB)], buf1, sem1)

        def compute(b, buf):
            accs = []
            for v in range(CB // L):
                accs.append(jnp.zeros((L,), jnp.float32))
            for k in range(K):
                wk = wbt_v[koff + k, :]
                for v in range(CB // L):
                    accs[v] = accs[v] + wk * buf[k, pl.ds(v * L, L)]
            for v in range(CB // L):
                out_v[pl.ds(v * L, L)] = accs[v]
            pltpu.sync_copy(
                out_v, out_hbm.at[pl.ds(base + b * CB, CB)])

        def body(g, carry):
            b0 = 2 * g
            pltpu.make_async_copy(
                src_hbm.at[:, pl.ds(base, CB)], buf0, sem0).wait()
            compute(b0, buf0)

            @pl.when(b0 + 2 < SC_BLKS)
            def _():
                pltpu.async_copy(
                    src_hbm.at[:, pl.ds(base + (b0 + 2) * CB, CB)],
                    buf0, sem0)

            pltpu.make_async_copy(
                src_hbm.at[:, pl.ds(base, CB)], buf1, sem1).wait()
            compute(b0 + 1, buf1)

            @pl.when(b0 + 3 < SC_BLKS)
            def _():
                pltpu.async_copy(
                    src_hbm.at[:, pl.ds(base + (b0 + 3) * CB, CB)],
                    buf1, sem1)
            return carry

        lax.fori_loop(0, SC_BLKS // 2, body, 0)

    table_loop(ut_hbm, su_hbm, 0)
    table_loop(it_hbm, si_hbm, K)


_mesh = plsc.VectorSubcoreMesh(
    core_axis_name="c", subcore_axis_name="s", num_cores=2, num_subcores=16)

_sc_scores_call = pl.kernel(
    _sc_scores_body,
    out_type=[
        jax.ShapeDtypeStruct((NS,), jnp.float32),
        jax.ShapeDtypeStruct((NS,), jnp.float32),
    ],
    mesh=_mesh,
    compiler_params=pltpu.CompilerParams(
        needs_layout_passes=False, use_tc_tiling_on_sc=True),
    scratch_types=[
        pltpu.VMEM((K, CB), jnp.float32),
        pltpu.VMEM((K, CB), jnp.float32),
        pltpu.VMEM((2 * K, L), jnp.float32),
        pltpu.VMEM((CB,), jnp.float32),
        pltpu.SemaphoreType.DMA,
        pltpu.SemaphoreType.DMA,
    ],
)


def _scores_body(ut_ref, it_ref, wu_ref, wi_ref, su_ref, si_ref):
    su_ref[...] = jnp.sum(ut_ref[...] * wu_ref[...], axis=0)
    si_ref[...] = jnp.sum(it_ref[...] * wi_ref[...], axis=0)


_scores_call = pl.pallas_call(
    _scores_body,
    grid=(pl.cdiv(NT, BN),),
    in_specs=[
        pl.BlockSpec((K, BN), lambda n: (0, n + TC_OFF)),
        pl.BlockSpec((K, BN), lambda n: (0, n + TC_OFF)),
        pl.BlockSpec((K, 1), lambda n: (0, 0)),
        pl.BlockSpec((K, 1), lambda n: (0, 0)),
    ],
    out_specs=[
        pl.BlockSpec((BN,), lambda n: (n,)),
        pl.BlockSpec((BN,), lambda n: (n,)),
    ],
    out_shape=[
        jax.ShapeDtypeStruct((NT,), jnp.float32),
        jax.ShapeDtypeStruct((NT,), jnp.float32),
    ],
)


def _gather_body(x_hbm, sus_hbm, sut_hbm, sis_hbm, sit_hbm, bias_hbm,
                 out_hbm, x_v, us_v, ut_v, is_v, it_v, sv_v, bias_v, out_v,
                 sem_u, sem_i):
    c = lax.axis_index("c")
    s = lax.axis_index("s")
    wid = s * 2 + c
    base = wid * BPW

    pltpu.sync_copy(bias_hbm, bias_v)
    pltpu.sync_copy(x_hbm.at[pl.ds(2 * base, 2 * BPW)], x_v)

    def deint(g, carry):
        jl2 = 2 * (g * L + lax.iota(jnp.int32, L))
        u = plsc.load_gather(x_v, [jl2])
        i = plsc.load_gather(x_v, [jl2 + 1])
        ch = g // (CH // L)
        off = (g % (CH // L)) * L
        us_v[ch, pl.ds(off, L)] = jnp.minimum(u, NS - 1)
        ut_v[ch, pl.ds(off, L)] = jnp.maximum(u - NS, 0)
        is_v[ch, pl.ds(off, L)] = jnp.minimum(i, NS - 1)
        it_v[ch, pl.ds(off, L)] = jnp.maximum(i - NS, 0)
        return carry

    lax.fori_loop(0, BPW // L, deint, 0)

    copies = []
    for ci in range(NCH):
        copies.append(pltpu.async_copy(
            sus_hbm.at[us_v.at[ci]],
            sv_v.at[pl.ds(ci * CH, CH)], sem_u))
        copies.append(pltpu.async_copy(
            sut_hbm.at[ut_v.at[ci]],
            sv_v.at[pl.ds(BPW + ci * CH, CH)], sem_u))
        copies.append(pltpu.async_copy(
            sis_hbm.at[is_v.at[ci]],
            sv_v.at[pl.ds(2 * BPW + ci * CH, CH)], sem_i))
        copies.append(pltpu.async_copy(
            sit_hbm.at[it_v.at[ci]],
            sv_v.at[pl.ds(3 * BPW + ci * CH, CH)], sem_i))
    for cp in copies:
        cp.wait()

    bias = bias_v[pl.ds(0, L)]

    def out_pass(g, carry):
        jl2 = 2 * (g * L + lax.iota(jnp.int32, L))
        u = plsc.load_gather(x_v, [jl2])
        i = plsc.load_gather(x_v, [jl2 + 1])
        zu = jnp.where(u < NS,
                       sv_v[pl.ds(g * L, L)],
                       sv_v[pl.ds(BPW + g * L, L)])
        zi = jnp.where(i < NS,
                       sv_v[pl.ds(2 * BPW + g * L, L)],
                       sv_v[pl.ds(3 * BPW + g * L, L)])
        z = zu + zi + bias
        out_v[pl.ds(g * L, L)] = 1.0 / (1.0 + jnp.exp(-z))
        return carry

    lax.fori_loop(0, BPW // L, out_pass, 0)
    pltpu.sync_copy(out_v, out_hbm.at[pl.ds(base, BPW)])


_gather_call = pl.kernel(
    _gather_body,
    out_type=jax.ShapeDtypeStruct((B,), jnp.float32),
    mesh=_mesh,
    compiler_params=pltpu.CompilerParams(
        needs_layout_passes=False, use_tc_tiling_on_sc=False),
    scratch_types=[
        pltpu.VMEM((2 * BPW,), jnp.int32),
        pltpu.VMEM((NCH, CH), jnp.int32),
        pltpu.VMEM((NCH, CH), jnp.int32),
        pltpu.VMEM((NCH, CH), jnp.int32),
        pltpu.VMEM((NCH, CH), jnp.int32),
        pltpu.VMEM((4 * BPW,), jnp.float32),
        pltpu.VMEM((L,), jnp.float32),
        pltpu.VMEM((BPW,), jnp.float32),
        pltpu.SemaphoreType.DMA,
        pltpu.SemaphoreType.DMA,
    ],
)


@jax.jit
def kernel(x, user_table, item_table, W, b):
    wu = W[:K]
    wi = W[K:]
    wbt = jnp.tile(W.reshape(2 * K, 1), (1, L))
    ut_t = user_table.T
    it_t = item_table.T
    su_sc, si_sc = _sc_scores_call(ut_t, it_t, wbt)
    su_tc, si_tc = _scores_call(ut_t, it_t, wu, wi)
    bias_t = jnp.tile(b, (L,))
    return _gather_call(x.reshape(-1), su_sc, su_tc, si_sc, si_tc, bias_t)


# speedup vs baseline: 1.3138x; 1.3138x over previous
import jax
import jax.numpy as jnp
from jax import lax
from jax.experimental import pallas as pl
from jax.experimental.pallas import tpu as pltpu
from jax.experimental.pallas import tpu_sc as plsc

B = 16384
K = 64
N = 1000000
BN = 16384
NW = 32
BPW = B // NW
NCH = 4
CH = BPW // NCH
L = 16

CB = 128
SC_BLKS = 16
NS = NW * SC_BLKS * CB
NT = N - NS
TC_OFF = NS // BN
assert NS % BN == 0


def _sc_scores_body(ut_hbm, it_hbm, wbt_hbm, su_hbm, si_hbm,
                    buf0, buf1, wbt_v, out_v, sem0, sem1):
    c = lax.axis_index("c")
    s = lax.axis_index("s")
    wid = s * 2 + c
    base = wid * SC_BLKS * CB

    pltpu.sync_copy(wbt_hbm, wbt_v)

    def table_loop(src_hbm, out_hbm, koff):
        pltpu.async_copy(src_hbm.at[:, pl.ds(base, CB)], buf0, sem0)
        pltpu.async_copy(src_hbm.at[:, pl.ds(base + CB, CB)], buf1, sem1)

        def compute(b, buf):
            accs = []
            for v in range(CB // L):
                accs.append(jnp.zeros((L,), jnp.float32))
            for k in range(K):
                wk = wbt_v[koff + k, :]
                for v in range(CB // L):
                    accs[v] = accs[v] + wk * buf[k, pl.ds(v * L, L)]
            for v in range(CB // L):
                out_v[pl.ds(v * L, L)] = accs[v]
            pltpu.sync_copy(
                out_v, out_hbm.at[pl.ds(base + b * CB, CB)])

        def body(g, carry):
            b0 = 2 * g
            pltpu.make_async_copy(
                src_hbm.at[:, pl.ds(base, CB)], buf0, sem0).wait()
            compute(b0, buf0)

            @pl.when(b0 + 2 < SC_BLKS)
            def _():
                pltpu.async_copy(
                    src_hbm.at[:, pl.ds(base + (b0 + 2) * CB, CB)],
                    buf0, sem0)

            pltpu.make_async_copy(
                src_hbm.at[:, pl.ds(base, CB)], buf1, sem1).wait()
            compute(b0 + 1, buf1)

            @pl.when(b0 + 3 < SC_BLKS)
            def _():
                pltpu.async_copy(
                    src_hbm.at[:, pl.ds(base + (b0 + 3) * CB, CB)],
                    buf1, sem1)
            return carry

        lax.fori_loop(0, SC_BLKS // 2, body, 0)

    table_loop(ut_hbm, su_hbm, 0)
    table_loop(it_hbm, si_hbm, K)


_mesh = plsc.VectorSubcoreMesh(
    core_axis_name="c", subcore_axis_name="s", num_cores=2, num_subcores=16)

_sc_scores_call = pl.kernel(
    _sc_scores_body,
    out_type=[
        jax.ShapeDtypeStruct((NS,), jnp.float32),
        jax.ShapeDtypeStruct((NS,), jnp.float32),
    ],
    mesh=_mesh,
    compiler_params=pltpu.CompilerParams(
        needs_layout_passes=False, use_tc_tiling_on_sc=True),
    scratch_types=[
        pltpu.VMEM((K, CB), jnp.float32),
        pltpu.VMEM((K, CB), jnp.float32),
        pltpu.VMEM((2 * K, L), jnp.float32),
        pltpu.VMEM((CB,), jnp.float32),
        pltpu.SemaphoreType.DMA,
        pltpu.SemaphoreType.DMA,
    ],
)


def _scores_body(ut_ref, it_ref, wu_ref, wi_ref, su_ref, si_ref):
    su_ref[...] = jnp.sum(ut_ref[...] * wu_ref[...], axis=0)
    si_ref[...] = jnp.sum(it_ref[...] * wi_ref[...], axis=0)


_scores_call = pl.pallas_call(
    _scores_body,
    grid=(pl.cdiv(NT, BN),),
    in_specs=[
        pl.BlockSpec((K, BN), lambda n: (0, n + TC_OFF)),
        pl.BlockSpec((K, BN), lambda n: (0, n + TC_OFF)),
        pl.BlockSpec((K, 1), lambda n: (0, 0)),
        pl.BlockSpec((K, 1), lambda n: (0, 0)),
    ],
    out_specs=[
        pl.BlockSpec((BN,), lambda n: (n,)),
        pl.BlockSpec((BN,), lambda n: (n,)),
    ],
    out_shape=[
        jax.ShapeDtypeStruct((NT,), jnp.float32),
        jax.ShapeDtypeStruct((NT,), jnp.float32),
    ],
)


def _gather_body(x_hbm, sus_hbm, sut_hbm, sis_hbm, sit_hbm, bias_hbm,
                 out_hbm, x_v, us_v, ut_v, is_v, it_v, sv_v, bias_v, out_v,
                 sem_u, sem_i):
    c = lax.axis_index("c")
    s = lax.axis_index("s")
    wid = s * 2 + c
    base = wid * BPW

    pltpu.sync_copy(bias_hbm, bias_v)
    pltpu.sync_copy(x_hbm.at[pl.ds(2 * base, 2 * BPW)], x_v)

    def deint(g, carry):
        jl2 = 2 * (g * L + lax.iota(jnp.int32, L))
        u = plsc.load_gather(x_v, [jl2])
        i = plsc.load_gather(x_v, [jl2 + 1])
        ch = g // (CH // L)
        off = (g % (CH // L)) * L
        us_v[ch, pl.ds(off, L)] = jnp.minimum(u, NS - 1)
        ut_v[ch, pl.ds(off, L)] = jnp.maximum(u - NS, 0)
        is_v[ch, pl.ds(off, L)] = jnp.minimum(i, NS - 1)
        it_v[ch, pl.ds(off, L)] = jnp.maximum(i - NS, 0)
        return carry

    lax.fori_loop(0, BPW // L, deint, 0)

    copies = []
    for ci in range(NCH):
        copies.append(pltpu.async_copy(
            sus_hbm.at[us_v.at[ci]],
            sv_v.at[pl.ds(ci * CH, CH)], sem_u))
        copies.append(pltpu.async_copy(
            sut_hbm.at[ut_v.at[ci]],
            sv_v.at[pl.ds(BPW + ci * CH, CH)], sem_u))
        copies.append(pltpu.async_copy(
            sis_hbm.at[is_v.at[ci]],
            sv_v.at[pl.ds(2 * BPW + ci * CH, CH)], sem_i))
        copies.append(pltpu.async_copy(
            sit_hbm.at[it_v.at[ci]],
            sv_v.at[pl.ds(3 * BPW + ci * CH, CH)], sem_i))
    for cp in copies:
        cp.wait()

    bias = bias_v[pl.ds(0, L)]

    def out_pass(g, carry):
        jl2 = 2 * (g * L + lax.iota(jnp.int32, L))
        u = plsc.load_gather(x_v, [jl2])
        i = plsc.load_gather(x_v, [jl2 + 1])
        zu = jnp.where(u < NS,
                       sv_v[pl.ds(g * L, L)],
                       sv_v[pl.ds(BPW + g * L, L)])
        zi = jnp.where(i < NS,
                       sv_v[pl.ds(2 * BPW + g * L, L)],
                       sv_v[pl.ds(3 * BPW + g * L, L)])
        z = zu + zi + bias
        out_v[pl.ds(g * L, L)] = 1.0 / (1.0 + jnp.exp(-z))
        return carry

    lax.fori_loop(0, BPW // L, out_pass, 0)
    pltpu.sync_copy(out_v, out_hbm.at[pl.ds(base, BPW)])


_gather_call = pl.kernel(
    _gather_body,
    out_type=jax.ShapeDtypeStruct((B,), jnp.float32),
    mesh=_mesh,
    compiler_params=pltpu.CompilerParams(
        needs_layout_passes=False, use_tc_tiling_on_sc=False),
    scratch_types=[
        pltpu.VMEM((2 * BPW,), jnp.int32),
        pltpu.VMEM((NCH, CH), jnp.int32),
        pltpu.VMEM((NCH, CH), jnp.int32),
        pltpu.VMEM((NCH, CH), jnp.int32),
        pltpu.VMEM((NCH, CH), jnp.int32),
        pltpu.VMEM((4 * BPW,), jnp.float32),
        pltpu.VMEM((L,), jnp.float32),
        pltpu.VMEM((BPW,), jnp.float32),
        pltpu.SemaphoreType.DMA,
        pltpu.SemaphoreType.DMA,
    ],
)


@jax.jit
def kernel(x, user_table, item_table, W, b):
    wu = W[:K]
    wi = W[K:]
    wbt = jnp.tile(W.reshape(2 * K, 1), (1, L))
    ut_t = user_table.T
    it_t = item_table.T
    su_sc, si_sc = _sc_scores_call(ut_t, it_t, wbt)
    su_tc, si_tc = _scores_call(ut_t, it_t, wu, wi)
    bias_t = jnp.tile(b, (L,))
    return _gather_call(x.reshape(-1), su_sc, su_tc, si_sc, si_tc, bias_t)


# speedup vs baseline: 1.8137x; 1.3805x over previous
import jax
import jax.numpy as jnp
from jax import lax
from jax.experimental import pallas as pl
from jax.experimental.pallas import tpu as pltpu
from jax.experimental.pallas import tpu_sc as plsc

B = 16384
K = 64
N = 1000000
BN = 16384
NW = 32
BPW = B // NW
NCH = 4
CH = BPW // NCH
L = 16


def _scores_body(ut_ref, it_ref, wu_ref, wi_ref, su_ref, si_ref):
    su_ref[...] = jnp.sum(ut_ref[...] * wu_ref[...], axis=0)
    si_ref[...] = jnp.sum(it_ref[...] * wi_ref[...], axis=0)


_scores_call = pl.pallas_call(
    _scores_body,
    grid=(pl.cdiv(N, BN),),
    in_specs=[
        pl.BlockSpec((K, BN), lambda n: (0, n)),
        pl.BlockSpec((K, BN), lambda n: (0, n)),
        pl.BlockSpec((K, 1), lambda n: (0, 0)),
        pl.BlockSpec((K, 1), lambda n: (0, 0)),
    ],
    out_specs=[
        pl.BlockSpec((BN,), lambda n: (n,)),
        pl.BlockSpec((BN,), lambda n: (n,)),
    ],
    out_shape=[
        jax.ShapeDtypeStruct((N,), jnp.float32),
        jax.ShapeDtypeStruct((N,), jnp.float32),
    ],
)


def _gather_body(x_hbm, su_hbm, si_hbm, bias_hbm, out_hbm,
                 x_v, uidx_v, iidx_v, sv_v, bias_v, out_v, sem_u, sem_i):
    c = lax.axis_index("c")
    s = lax.axis_index("s")
    wid = s * 2 + c
    base = wid * BPW

    pltpu.sync_copy(bias_hbm, bias_v)
    pltpu.sync_copy(x_hbm.at[pl.ds(2 * base, 2 * BPW)], x_v)

    def deint(g, carry):
        jl2 = 2 * (g * L + lax.iota(jnp.int32, L))
        u = plsc.load_gather(x_v, [jl2])
        i = plsc.load_gather(x_v, [jl2 + 1])
        ch = g // (CH // L)
        off = (g % (CH // L)) * L
        uidx_v[ch, pl.ds(off, L)] = u
        iidx_v[ch, pl.ds(off, L)] = i
        return carry

    lax.fori_loop(0, BPW // L, deint, 0)

    copies = []
    for ci in range(NCH):
        copies.append(pltpu.async_copy(
            su_hbm.at[uidx_v.at[ci]],
            sv_v.at[pl.ds(ci * CH, CH)], sem_u))
        copies.append(pltpu.async_copy(
            si_hbm.at[iidx_v.at[ci]],
            sv_v.at[pl.ds(BPW + ci * CH, CH)], sem_i))
    for cp in copies:
        cp.wait()

    bias = bias_v[pl.ds(0, L)]
    for q in range(BPW // L):
        z = sv_v[pl.ds(q * L, L)] + sv_v[pl.ds(BPW + q * L, L)] + bias
        out_v[pl.ds(q * L, L)] = 1.0 / (1.0 + jnp.exp(-z))
    pltpu.sync_copy(out_v, out_hbm.at[pl.ds(base, BPW)])


_mesh = plsc.VectorSubcoreMesh(
    core_axis_name="c", subcore_axis_name="s", num_cores=2, num_subcores=16)

_gather_call = pl.kernel(
    _gather_body,
    out_type=jax.ShapeDtypeStruct((B,), jnp.float32),
    mesh=_mesh,
    compiler_params=pltpu.CompilerParams(
        needs_layout_passes=False, use_tc_tiling_on_sc=False),
    scratch_types=[
        pltpu.VMEM((2 * BPW,), jnp.int32),
        pltpu.VMEM((NCH, CH), jnp.int32),
        pltpu.VMEM((NCH, CH), jnp.int32),
        pltpu.VMEM((2 * BPW,), jnp.float32),
        pltpu.VMEM((L,), jnp.float32),
        pltpu.VMEM((BPW,), jnp.float32),
        pltpu.SemaphoreType.DMA,
        pltpu.SemaphoreType.DMA,
    ],
)


@jax.jit
def kernel(x, user_table, item_table, W, b):
    wu = W[:K]
    wi = W[K:]
    su, si = _scores_call(user_table.T, item_table.T, wu, wi)
    bias_t = jnp.tile(b, (L,))
    return _gather_call(x.reshape(-1), su, si, bias_t)
